# Initial kernel scaffold; baseline (speedup 1.0000x reference)
import jax
import jax.numpy as jnp
from jax import lax
from jax.experimental import pallas as pl
from jax.experimental.pallas import tpu as pltpu
from jax.experimental.pallas import tpu_sc as plsc

N = 320000
D = 128
S = 10000
EPS = 1e-5

ROW_BLOCK = 2000
CHUNK = 128
NUM_CHUNKS = N // CHUNK
NC = 2
NS = 16
NW = NC * NS
ROWS_PER_SUB = S // NS
OUT_BLOCK = 2000


def _linear_ln_body(x_ref, wt_ref, b_ref, g_ref, bt_ref, h_ref):
    h = jnp.dot(x_ref[...], wt_ref[...], preferred_element_type=jnp.float32)
    h = h + b_ref[...]
    mu = jnp.mean(h, axis=-1, keepdims=True)
    var = jnp.mean((h - mu) ** 2, axis=-1, keepdims=True)
    h_ref[...] = (h - mu) * lax.rsqrt(var + EPS) * g_ref[...] + bt_ref[...]


def _linear_ln(x, wt, b2, g2, bt2):
    grid = (N // ROW_BLOCK,)
    return pl.pallas_call(
        _linear_ln_body,
        grid=grid,
        in_specs=[
            pl.BlockSpec((ROW_BLOCK, D), lambda i: (i, 0)),
            pl.BlockSpec((D, D), lambda i: (0, 0)),
            pl.BlockSpec((1, D), lambda i: (0, 0)),
            pl.BlockSpec((1, D), lambda i: (0, 0)),
            pl.BlockSpec((1, D), lambda i: (0, 0)),
        ],
        out_specs=pl.BlockSpec((ROW_BLOCK, D), lambda i: (i, 0)),
        out_shape=jax.ShapeDtypeStruct((N, D), jnp.float32),
    )(x, wt, b2, g2, bt2)


def _sc_body(h_hbm, b2d_hbm, zrow_hbm, zcnt_hbm, psum_hbm, cnt_hbm,
             acc, idx_v, rows_v, cnt_v):
    cid = lax.axis_index("c")
    sid = lax.axis_index("s")
    wid = cid * NS + sid

    pltpu.sync_copy(zrow_hbm, acc.at[pl.ds(sid * ROWS_PER_SUB, ROWS_PER_SUB)])
    pltpu.sync_copy(zcnt_hbm, cnt_v)
    plsc.subcore_barrier()

    ones = jnp.ones((16,), jnp.float32)

    def step(k, carry):
        chunk = wid + NW * k

        @pl.when(chunk < NUM_CHUNKS)
        def _():
            pltpu.sync_copy(b2d_hbm.at[chunk], idx_v.at[0])
            pltpu.sync_copy(h_hbm.at[pl.ds(chunk * CHUNK, CHUNK)], rows_v)
            pltpu.sync_copy(rows_v, acc.at[idx_v.at[0]], add=True)
            for t in range(CHUNK // 16):
                ids = idx_v[0, pl.ds(t * 16, 16)]
                plsc.addupdate_scatter(cnt_v, [ids], ones)

        return carry

    lax.fori_loop(0, (NUM_CHUNKS + NW - 1) // NW, step, 0)

    pltpu.sync_copy(cnt_v, cnt_hbm.at[wid])
    plsc.subcore_barrier()
    pltpu.sync_copy(acc.at[pl.ds(sid * ROWS_PER_SUB, ROWS_PER_SUB)],
                    psum_hbm.at[cid, pl.ds(sid * ROWS_PER_SUB, ROWS_PER_SUB)])


def _segment_sums(h, b2d, zrow, zcnt):
    mesh = plsc.VectorSubcoreMesh(core_axis_name="c", subcore_axis_name="s")
    return pl.kernel(
        _sc_body,
        out_type=[
            jax.ShapeDtypeStruct((NC, S, D), jnp.float32),
            jax.ShapeDtypeStruct((NW, S), jnp.float32),
        ],
        mesh=mesh,
        scratch_types=[
            pltpu.VMEM_SHARED((S, D), jnp.float32),
            pltpu.VMEM((1, CHUNK), jnp.int32),
            pltpu.VMEM((CHUNK, D), jnp.float32),
            pltpu.VMEM((S,), jnp.float32),
        ],
    )(h, b2d, zrow, zcnt)


def _combine_body(p_ref, c_ref, o_ref):
    cnt = jnp.maximum(jnp.sum(c_ref[...], axis=0), 1.0)
    o_ref[...] = (p_ref[0] + p_ref[1]) / cnt[:, None]


def _combine(psum, cnt):
    grid = (S // OUT_BLOCK,)
    return pl.pallas_call(
        _combine_body,
        grid=grid,
        in_specs=[
            pl.BlockSpec((NC, OUT_BLOCK, D), lambda i: (0, i, 0)),
            pl.BlockSpec((NW, OUT_BLOCK), lambda i: (0, i)),
        ],
        out_specs=pl.BlockSpec((OUT_BLOCK, D), lambda i: (i, 0)),
        out_shape=jax.ShapeDtypeStruct((S, D), jnp.float32),
    )(psum, cnt)


def kernel(x, batch, W, b, gamma, beta):
    wt = W.T
    b2 = b.reshape(1, D)
    g2 = gamma.reshape(1, D)
    bt2 = beta.reshape(1, D)
    h = _linear_ln(x, wt, b2, g2, bt2)
    b2d = batch.reshape(NUM_CHUNKS, CHUNK)
    zrow = jnp.zeros((ROWS_PER_SUB, D), jnp.float32)
    zcnt = jnp.zeros((S,), jnp.float32)
    psum, cnt = _segment_sums(h, b2d, zrow, zcnt)
    return _combine(psum, cnt)


# speedup vs baseline: 3.5893x; 3.5893x over previous
import jax
import jax.numpy as jnp
from jax import lax
from jax.experimental import pallas as pl
from jax.experimental.pallas import tpu as pltpu
from jax.experimental.pallas import tpu_sc as plsc

N = 320000
D = 128
S = 10000
EPS = 1e-5

ROW_BLOCK = 2000
CHUNK = 128
NUM_CHUNKS = N // CHUNK
NC = 2
NS = 16
NW = NC * NS
SP = 10240
ROWS_PER_SUB = SP // NS


def _linear_ln_body(x_ref, wt_ref, b_ref, g_ref, bt_ref, h_ref):
    h = jnp.dot(x_ref[...], wt_ref[...], preferred_element_type=jnp.float32)
    h = h + b_ref[...]
    mu = jnp.mean(h, axis=-1, keepdims=True)
    var = jnp.mean((h - mu) ** 2, axis=-1, keepdims=True)
    h_ref[...] = (h - mu) * lax.rsqrt(var + EPS) * g_ref[...] + bt_ref[...]


def _linear_ln(x, wt, b2, g2, bt2):
    grid = (N // ROW_BLOCK,)
    return pl.pallas_call(
        _linear_ln_body,
        grid=grid,
        in_specs=[
            pl.BlockSpec((ROW_BLOCK, D), lambda i: (i, 0)),
            pl.BlockSpec((D, D), lambda i: (0, 0)),
            pl.BlockSpec((1, D), lambda i: (0, 0)),
            pl.BlockSpec((1, D), lambda i: (0, 0)),
            pl.BlockSpec((1, D), lambda i: (0, 0)),
        ],
        out_specs=pl.BlockSpec((ROW_BLOCK, D), lambda i: (i, 0)),
        out_shape=jax.ShapeDtypeStruct((N, D), jnp.float32),
    )(x, wt, b2, g2, bt2)


def _sc_body(h_hbm, b2d_hbm, zrow_hbm, zcnt_hbm, ones_hbm, psum_hbm, cnt_hbm,
             acc, cacc, idx_v, rows_v, ones_v, zc16_v):
    cid = lax.axis_index("c")
    sid = lax.axis_index("s")
    wid = cid * NS + sid
    base = sid * ROWS_PER_SUB

    pltpu.sync_copy(zrow_hbm, rows_v)
    pltpu.sync_copy(zcnt_hbm, zc16_v)
    pltpu.sync_copy(ones_hbm, ones_v)
    for j in range(ROWS_PER_SUB // CHUNK):
        pltpu.sync_copy(rows_v, acc.at[pl.ds(base + j * CHUNK, CHUNK)])
        pltpu.sync_copy(zc16_v, cacc.at[pl.ds(base + j * CHUNK, CHUNK)])
    plsc.subcore_barrier()

    def step(k, carry):
        chunk = wid + NW * k

        @pl.when(chunk < NUM_CHUNKS)
        def _():
            pltpu.sync_copy(b2d_hbm.at[chunk], idx_v.at[0])
            pltpu.sync_copy(h_hbm.at[pl.ds(chunk * CHUNK, CHUNK)], rows_v)
            pltpu.sync_copy(rows_v, acc.at[idx_v.at[0]], add=True)
            pltpu.sync_copy(ones_v, cacc.at[idx_v.at[0]], add=True)

        return carry

    lax.fori_loop(0, (NUM_CHUNKS + NW - 1) // NW, step, 0)

    plsc.subcore_barrier()
    for j in range(ROWS_PER_SUB // CHUNK):
        pltpu.sync_copy(acc.at[pl.ds(base + j * CHUNK, CHUNK)], rows_v)
        pltpu.sync_copy(rows_v, psum_hbm.at[cid, pl.ds(base + j * CHUNK, CHUNK)])
        pltpu.sync_copy(cacc.at[pl.ds(base + j * CHUNK, CHUNK)], zc16_v)
        pltpu.sync_copy(zc16_v, cnt_hbm.at[cid, pl.ds(base + j * CHUNK, CHUNK)])


def _segment_sums(h, b2d, zrow, zcnt, ones):
    mesh = plsc.VectorSubcoreMesh(core_axis_name="c", subcore_axis_name="s")
    return pl.kernel(
        _sc_body,
        out_type=[
            jax.ShapeDtypeStruct((NC, SP, D), jnp.float32),
            jax.ShapeDtypeStruct((NC, SP, 16), jnp.float32),
        ],
        mesh=mesh,
        compiler_params=pltpu.CompilerParams(use_tc_tiling_on_sc=False),
        scratch_types=[
            pltpu.VMEM_SHARED((SP, D), jnp.float32),
            pltpu.VMEM_SHARED((SP, 16), jnp.float32),
            pltpu.VMEM((1, CHUNK), jnp.int32),
            pltpu.VMEM((CHUNK, D), jnp.float32),
            pltpu.VMEM((CHUNK, 16), jnp.float32),
            pltpu.VMEM((CHUNK, 16), jnp.float32),
        ],
    )(h, b2d, zrow, zcnt, ones)


def _combine_body(p_ref, c_ref, o_ref):
    cnt = jnp.maximum(c_ref[0, :S, 0:1] + c_ref[1, :S, 0:1], 1.0)
    o_ref[...] = (p_ref[0, :S] + p_ref[1, :S]) / cnt


def _combine(psum, cnt):
    return pl.pallas_call(
        _combine_body,
        out_shape=jax.ShapeDtypeStruct((S, D), jnp.float32),
    )(psum, cnt)


def kernel(x, batch, W, b, gamma, beta):
    wt = W.T
    b2 = b.reshape(1, D)
    g2 = gamma.reshape(1, D)
    bt2 = beta.reshape(1, D)
    h = _linear_ln(x, wt, b2, g2, bt2)
    b2d = batch.reshape(NUM_CHUNKS, CHUNK)
    zrow = jnp.zeros((CHUNK, D), jnp.float32)
    zcnt = jnp.zeros((CHUNK, 16), jnp.float32)
    ones = jnp.ones((CHUNK, 16), jnp.float32)
    psum, cnt = _segment_sums(h, b2d, zrow, zcnt, ones)
    return _combine(psum, cnt)


# speedup vs baseline: 4.4545x; 1.2410x over previous
import jax
import jax.numpy as jnp
from jax import lax
from jax.experimental import pallas as pl
from jax.experimental.pallas import tpu as pltpu
from jax.experimental.pallas import tpu_sc as plsc

N = 320000
D = 128
S = 10000
EPS = 1e-5

ROW_BLOCK = 2000
CHUNK = 128
NUM_CHUNKS = N // CHUNK
NC = 2
NS = 16
NW = NC * NS
SP = 10240
ROWS_PER_SUB = SP // NS


def _linear_ln_body(x_ref, wt_ref, b_ref, g_ref, bt_ref, h_ref):
    h = jnp.dot(x_ref[...], wt_ref[...], preferred_element_type=jnp.float32)
    h = h + b_ref[...]
    mu = jnp.mean(h, axis=-1, keepdims=True)
    var = jnp.mean((h - mu) ** 2, axis=-1, keepdims=True)
    h_ref[...] = (h - mu) * lax.rsqrt(var + EPS) * g_ref[...] + bt_ref[...]


def _linear_ln(x, wt, b2, g2, bt2):
    grid = (N // ROW_BLOCK,)
    return pl.pallas_call(
        _linear_ln_body,
        grid=grid,
        in_specs=[
            pl.BlockSpec((ROW_BLOCK, D), lambda i: (i, 0)),
            pl.BlockSpec((D, D), lambda i: (0, 0)),
            pl.BlockSpec((1, D), lambda i: (0, 0)),
            pl.BlockSpec((1, D), lambda i: (0, 0)),
            pl.BlockSpec((1, D), lambda i: (0, 0)),
        ],
        out_specs=pl.BlockSpec((ROW_BLOCK, D), lambda i: (i, 0)),
        out_shape=jax.ShapeDtypeStruct((N, D), jnp.float32),
    )(x, wt, b2, g2, bt2)


BLK = 128
SUB = BLK // CHUNK
NBLK = N // BLK
T_OUTER = 40


def _sc_body(h_hbm, b2d_hbm, zrow_hbm, zcnt_hbm, ones_hbm, psum_hbm, cnt_hbm,
             acc, cacc, idx_v, rows_v, ones_v, zc16_v,
             lsem0, lsem1, ssem0, ssem1):
    cid = lax.axis_index("c")
    sid = lax.axis_index("s")
    wid = cid * NS + sid
    base = sid * ROWS_PER_SUB
    lsem = (lsem0, lsem1)
    ssem = (ssem0, ssem1)

    start = 78 * wid + jnp.minimum(wid, 4)
    nblk = 78 + jnp.where(wid < 4, 1, 0)

    pltpu.sync_copy(zrow_hbm, rows_v.at[0])
    pltpu.sync_copy(zcnt_hbm, zc16_v)
    pltpu.sync_copy(ones_hbm, ones_v)
    for j in range(ROWS_PER_SUB // CHUNK):
        pltpu.sync_copy(rows_v.at[0], acc.at[pl.ds(base + j * CHUNK, CHUNK)])
    for j in range(ROWS_PER_SUB // CHUNK):
        pltpu.sync_copy(zc16_v, cacc.at[pl.ds(base + j * CHUNK, CHUNK)])
    plsc.subcore_barrier()

    def issue_load(blk, buf):
        pltpu.async_copy(b2d_hbm.at[blk], idx_v.at[buf], lsem[buf])
        pltpu.async_copy(h_hbm.at[pl.ds(blk * BLK, BLK)], rows_v.at[buf],
                         lsem[buf])

    def wait_load(blk, buf):
        pltpu.make_async_copy(b2d_hbm.at[blk], idx_v.at[buf], lsem[buf]).wait()
        pltpu.make_async_copy(h_hbm.at[pl.ds(blk * BLK, BLK)], rows_v.at[buf],
                              lsem[buf]).wait()

    def issue_scat(buf):
        for j in range(SUB):
            pltpu.async_copy(rows_v.at[buf, pl.ds(j * CHUNK, CHUNK)],
                             acc.at[idx_v.at[buf, j]], ssem[buf], add=True)
            pltpu.async_copy(ones_v, cacc.at[idx_v.at[buf, j]], ssem[buf],
                             add=True)

    def wait_scat(buf):
        for j in range(SUB):
            pltpu.make_async_copy(rows_v.at[buf, pl.ds(j * CHUNK, CHUNK)],
                                  acc.at[idx_v.at[buf, j]], ssem[buf]).wait()
            pltpu.make_async_copy(ones_v, cacc.at[idx_v.at[buf, j]],
                                  ssem[buf]).wait()

    issue_load(start, 0)

    def t_body(t, carry):
        for half in range(2):
            k = 2 * t + half
            buf = half

            @pl.when(k < nblk)
            def _():
                wait_load(start + k, buf)
                issue_scat(buf)

            @pl.when(k + 1 < nblk)
            def _():
                @pl.when(k >= 1)
                def __():
                    wait_scat(1 - buf)

                issue_load(start + k + 1, 1 - buf)

        return carry

    lax.fori_loop(0, T_OUTER, t_body, 0)
    wait_scat(0)
    wait_scat(1)
    plsc.subcore_barrier()

    for j in range(ROWS_PER_SUB // CHUNK):
        pltpu.sync_copy(acc.at[pl.ds(base + j * CHUNK, CHUNK)],
                        rows_v.at[j % 2])
        pltpu.sync_copy(rows_v.at[j % 2],
                        psum_hbm.at[cid, pl.ds(base + j * CHUNK, CHUNK)])
    for j in range(ROWS_PER_SUB // CHUNK):
        pltpu.sync_copy(cacc.at[pl.ds(base + j * CHUNK, CHUNK)], zc16_v)
        pltpu.sync_copy(zc16_v, cnt_hbm.at[cid, pl.ds(base + j * CHUNK, CHUNK)])


def _segment_sums(h, b2d, zrow, zcnt, ones):
    mesh = plsc.VectorSubcoreMesh(core_axis_name="c", subcore_axis_name="s")
    return pl.kernel(
        _sc_body,
        out_type=[
            jax.ShapeDtypeStruct((NC, SP, D), jnp.float32),
            jax.ShapeDtypeStruct((NC, SP, 16), jnp.float32),
        ],
        mesh=mesh,
        compiler_params=pltpu.CompilerParams(use_tc_tiling_on_sc=False),
        scratch_types=[
            pltpu.VMEM_SHARED((SP, D), jnp.float32),
            pltpu.VMEM_SHARED((SP, 16), jnp.float32),
            pltpu.VMEM((2, SUB, CHUNK), jnp.int32),
            pltpu.VMEM((2, BLK, D), jnp.float32),
            pltpu.VMEM((CHUNK, 16), jnp.float32),
            pltpu.VMEM((CHUNK, 16), jnp.float32),
            pltpu.SemaphoreType.DMA,
            pltpu.SemaphoreType.DMA,
            pltpu.SemaphoreType.DMA,
            pltpu.SemaphoreType.DMA,
        ],
    )(h, b2d, zrow, zcnt, ones)


def _combine_body(p_ref, c_ref, o_ref):
    cnt = jnp.maximum(c_ref[0, :S, 0:1] + c_ref[1, :S, 0:1], 1.0)
    o_ref[...] = (p_ref[0, :S] + p_ref[1, :S]) / cnt


def _combine(psum, cnt):
    return pl.pallas_call(
        _combine_body,
        out_shape=jax.ShapeDtypeStruct((S, D), jnp.float32),
    )(psum, cnt)


def kernel(x, batch, W, b, gamma, beta):
    wt = W.T
    b2 = b.reshape(1, D)
    g2 = gamma.reshape(1, D)
    bt2 = beta.reshape(1, D)
    h = _linear_ln(x, wt, b2, g2, bt2)
    b2d = batch.reshape(NBLK, SUB, CHUNK)
    zrow = jnp.zeros((BLK, D), jnp.float32)
    zcnt = jnp.zeros((CHUNK, 16), jnp.float32)
    ones = jnp.ones((CHUNK, 16), jnp.float32)
    psum, cnt = _segment_sums(h, b2d, zrow, zcnt, ones)
    return _combine(psum, cnt)


# speedup vs baseline: 5.1483x; 1.1557x over previous
import jax
import jax.numpy as jnp
from jax import lax
from jax.experimental import pallas as pl
from jax.experimental.pallas import tpu as pltpu
from jax.experimental.pallas import tpu_sc as plsc

N = 320000
D = 128
S = 10000
EPS = 1e-5

ROW_BLOCK = 4000
CHUNK = 128
NUM_CHUNKS = N // CHUNK
NC = 2
NS = 16
NW = NC * NS
SP = 10240
ROWS_PER_SUB = SP // NS


def _linear_ln_body(x_ref, wt_ref, b_ref, g_ref, bt_ref, h_ref):
    h = jnp.dot(x_ref[...], wt_ref[...], preferred_element_type=jnp.float32)
    h = h + b_ref[...]
    mu = jnp.mean(h, axis=-1, keepdims=True)
    var = jnp.mean((h - mu) ** 2, axis=-1, keepdims=True)
    h_ref[...] = (h - mu) * lax.rsqrt(var + EPS) * g_ref[...] + bt_ref[...]


def _linear_ln(x, wt, b2, g2, bt2):
    grid = (N // ROW_BLOCK,)
    return pl.pallas_call(
        _linear_ln_body,
        grid=grid,
        in_specs=[
            pl.BlockSpec((ROW_BLOCK, D), lambda i: (i, 0)),
            pl.BlockSpec((D, D), lambda i: (0, 0)),
            pl.BlockSpec((1, D), lambda i: (0, 0)),
            pl.BlockSpec((1, D), lambda i: (0, 0)),
            pl.BlockSpec((1, D), lambda i: (0, 0)),
        ],
        out_specs=pl.BlockSpec((ROW_BLOCK, D), lambda i: (i, 0)),
        out_shape=jax.ShapeDtypeStruct((N, D), jnp.float32),
    )(x, wt, b2, g2, bt2)


BLK = 128
SUB = BLK // CHUNK
NBLK = N // BLK
T_OUTER = 40


def _sc_body(h_hbm, b2d_hbm, zrow_hbm, zcnt_hbm, ones_hbm, psum_hbm, cnt_hbm,
             acc, cacc, idx_v, rows_v, ones_v, zc16_v,
             lsem0, lsem1, ssem0, ssem1):
    cid = lax.axis_index("c")
    sid = lax.axis_index("s")
    wid = cid * NS + sid
    base = sid * ROWS_PER_SUB
    lsem = (lsem0, lsem1)
    ssem = (ssem0, ssem1)

    start = 78 * wid + jnp.minimum(wid, 4)
    nblk = 78 + jnp.where(wid < 4, 1, 0)

    pltpu.sync_copy(zrow_hbm, rows_v.at[0])
    pltpu.sync_copy(zcnt_hbm, zc16_v)
    pltpu.sync_copy(ones_hbm, ones_v)
    for j in range(ROWS_PER_SUB // CHUNK):
        pltpu.sync_copy(rows_v.at[0], acc.at[pl.ds(base + j * CHUNK, CHUNK)])
    for j in range(ROWS_PER_SUB // CHUNK):
        pltpu.sync_copy(zc16_v, cacc.at[pl.ds(base + j * CHUNK, CHUNK)])
    plsc.subcore_barrier()

    def issue_load(blk, buf):
        pltpu.async_copy(b2d_hbm.at[pl.ds(blk * CHUNK, CHUNK)],
                         idx_v.at[buf, 0], lsem[buf])
        pltpu.async_copy(h_hbm.at[pl.ds(blk * BLK, BLK)], rows_v.at[buf],
                         lsem[buf])

    def wait_load(blk, buf):
        pltpu.make_async_copy(b2d_hbm.at[pl.ds(blk * CHUNK, CHUNK)],
                              idx_v.at[buf, 0], lsem[buf]).wait()
        pltpu.make_async_copy(h_hbm.at[pl.ds(blk * BLK, BLK)], rows_v.at[buf],
                              lsem[buf]).wait()

    def issue_scat(buf):
        for j in range(SUB):
            pltpu.async_copy(rows_v.at[buf, pl.ds(j * CHUNK, CHUNK)],
                             acc.at[idx_v.at[buf, j]], ssem[buf], add=True)
            pltpu.async_copy(ones_v, cacc.at[idx_v.at[buf, j]], ssem[buf],
                             add=True)

    def wait_scat(buf):
        for j in range(SUB):
            pltpu.make_async_copy(rows_v.at[buf, pl.ds(j * CHUNK, CHUNK)],
                                  acc.at[idx_v.at[buf, j]], ssem[buf]).wait()
            pltpu.make_async_copy(ones_v, cacc.at[idx_v.at[buf, j]],
                                  ssem[buf]).wait()

    issue_load(start, 0)

    def t_body(t, carry):
        for half in range(2):
            k = 2 * t + half
            buf = half

            @pl.when(k < nblk)
            def _():
                wait_load(start + k, buf)
                issue_scat(buf)

            @pl.when(k + 1 < nblk)
            def _():
                @pl.when(k >= 1)
                def __():
                    wait_scat(1 - buf)

                issue_load(start + k + 1, 1 - buf)

        return carry

    lax.fori_loop(0, T_OUTER, t_body, 0)
    wait_scat(0)
    wait_scat(1)
    plsc.subcore_barrier()

    for j in range(ROWS_PER_SUB // CHUNK):
        pltpu.sync_copy(acc.at[pl.ds(base + j * CHUNK, CHUNK)],
                        rows_v.at[j % 2])
        pltpu.sync_copy(rows_v.at[j % 2],
                        psum_hbm.at[cid, pl.ds(base + j * CHUNK, CHUNK)])
    for j in range(ROWS_PER_SUB // CHUNK):
        pltpu.sync_copy(cacc.at[pl.ds(base + j * CHUNK, CHUNK)], zc16_v)
        pltpu.sync_copy(zc16_v, cnt_hbm.at[cid, pl.ds(base + j * CHUNK, CHUNK)])


def _segment_sums(h, b2d, zrow, zcnt, ones):
    mesh = plsc.VectorSubcoreMesh(core_axis_name="c", subcore_axis_name="s")
    return pl.kernel(
        _sc_body,
        out_type=[
            jax.ShapeDtypeStruct((NC, SP, D), jnp.float32),
            jax.ShapeDtypeStruct((NC, SP, 16), jnp.float32),
        ],
        mesh=mesh,
        compiler_params=pltpu.CompilerParams(use_tc_tiling_on_sc=False),
        scratch_types=[
            pltpu.VMEM_SHARED((SP, D), jnp.float32),
            pltpu.VMEM_SHARED((SP, 16), jnp.float32),
            pltpu.VMEM((2, SUB, CHUNK), jnp.int32),
            pltpu.VMEM((2, BLK, D), jnp.float32),
            pltpu.VMEM((CHUNK, 16), jnp.float32),
            pltpu.VMEM((CHUNK, 16), jnp.float32),
            pltpu.SemaphoreType.DMA,
            pltpu.SemaphoreType.DMA,
            pltpu.SemaphoreType.DMA,
            pltpu.SemaphoreType.DMA,
        ],
    )(h, b2d, zrow, zcnt, ones)


def _combine_body(p_ref, c_ref, o_ref):
    cnt = jnp.maximum(c_ref[0, :S, 0:1] + c_ref[1, :S, 0:1], 1.0)
    o_ref[...] = (p_ref[0, :S] + p_ref[1, :S]) / cnt


def _combine(psum, cnt):
    return pl.pallas_call(
        _combine_body,
        out_shape=jax.ShapeDtypeStruct((S, D), jnp.float32),
    )(psum, cnt)


def kernel(x, batch, W, b, gamma, beta):
    wt = W.T
    b2 = b.reshape(1, D)
    g2 = gamma.reshape(1, D)
    bt2 = beta.reshape(1, D)
    h = _linear_ln(x, wt, b2, g2, bt2)
    zrow = jnp.zeros((BLK, D), jnp.float32)
    zcnt = jnp.zeros((CHUNK, 16), jnp.float32)
    ones = jnp.ones((CHUNK, 16), jnp.float32)
    psum, cnt = _segment_sums(h, batch, zrow, zcnt, ones)
    return _combine(psum, cnt)


# speedup vs baseline: 5.6065x; 1.0890x over previous
import jax
import jax.numpy as jnp
from jax import lax
from jax.experimental import pallas as pl
from jax.experimental.pallas import tpu as pltpu
from jax.experimental.pallas import tpu_sc as plsc

N = 320000
D = 128
S = 10000
EPS = 1e-5

ROW_BLOCK = 8000
CHUNK = 128
NUM_CHUNKS = N // CHUNK
NC = 2
NS = 16
NW = NC * NS
SP = 10240
ROWS_PER_SUB = SP // NS


def _linear_ln_body(x_ref, wt_ref, b_ref, g_ref, bt_ref, h_ref):
    h = jnp.dot(x_ref[...], wt_ref[...], preferred_element_type=jnp.float32)
    h = h + b_ref[...]
    mu = jnp.mean(h, axis=-1, keepdims=True)
    var = jnp.mean((h - mu) ** 2, axis=-1, keepdims=True)
    h_ref[...] = (h - mu) * lax.rsqrt(var + EPS) * g_ref[...] + bt_ref[...]


def _linear_ln(x, wt, b2, g2, bt2):
    grid = (N // ROW_BLOCK,)
    return pl.pallas_call(
        _linear_ln_body,
        grid=grid,
        in_specs=[
            pl.BlockSpec((ROW_BLOCK, D), lambda i: (i, 0)),
            pl.BlockSpec((D, D), lambda i: (0, 0)),
            pl.BlockSpec((1, D), lambda i: (0, 0)),
            pl.BlockSpec((1, D), lambda i: (0, 0)),
            pl.BlockSpec((1, D), lambda i: (0, 0)),
        ],
        out_specs=pl.BlockSpec((ROW_BLOCK, D), lambda i: (i, 0)),
        out_shape=jax.ShapeDtypeStruct((N, D), jnp.float32),
    )(x, wt, b2, g2, bt2)


BLK = 128
SUB = BLK // CHUNK
NBLK = N // BLK
T_OUTER = 40


def _sc_body(h_hbm, b2d_hbm, zrow_hbm, zcnt_hbm, ones_hbm, psum_hbm, cnt_hbm,
             acc, cacc, idx_v, rows_v, ones_v, zc16_v,
             lsem0, lsem1, ssem0, ssem1):
    cid = lax.axis_index("c")
    sid = lax.axis_index("s")
    wid = cid * NS + sid
    base = sid * ROWS_PER_SUB
    lsem = (lsem0, lsem1)
    ssem = (ssem0, ssem1)

    start = 78 * wid + jnp.minimum(wid, 4)
    nblk = 78 + jnp.where(wid < 4, 1, 0)

    pltpu.sync_copy(zrow_hbm, rows_v.at[0])
    pltpu.sync_copy(zcnt_hbm, zc16_v)
    pltpu.sync_copy(ones_hbm, ones_v)
    for j in range(ROWS_PER_SUB // CHUNK):
        pltpu.sync_copy(rows_v.at[0], acc.at[pl.ds(base + j * CHUNK, CHUNK)])
    for j in range(ROWS_PER_SUB // CHUNK):
        pltpu.sync_copy(zc16_v, cacc.at[pl.ds(base + j * CHUNK, CHUNK)])
    plsc.subcore_barrier()

    def issue_load(blk, buf):
        pltpu.async_copy(b2d_hbm.at[pl.ds(blk * CHUNK, CHUNK)],
                         idx_v.at[buf, 0], lsem[buf])
        pltpu.async_copy(h_hbm.at[pl.ds(blk * BLK, BLK)], rows_v.at[buf],
                         lsem[buf])

    def wait_load(blk, buf):
        pltpu.make_async_copy(b2d_hbm.at[pl.ds(blk * CHUNK, CHUNK)],
                              idx_v.at[buf, 0], lsem[buf]).wait()
        pltpu.make_async_copy(h_hbm.at[pl.ds(blk * BLK, BLK)], rows_v.at[buf],
                              lsem[buf]).wait()

    def issue_scat(buf):
        for j in range(SUB):
            pltpu.async_copy(rows_v.at[buf, pl.ds(j * CHUNK, CHUNK)],
                             acc.at[idx_v.at[buf, j]], ssem[buf], add=True)
            pltpu.async_copy(ones_v, cacc.at[idx_v.at[buf, j]], ssem[buf],
                             add=True)

    def wait_scat(buf):
        for j in range(SUB):
            pltpu.make_async_copy(rows_v.at[buf, pl.ds(j * CHUNK, CHUNK)],
                                  acc.at[idx_v.at[buf, j]], ssem[buf]).wait()
            pltpu.make_async_copy(ones_v, cacc.at[idx_v.at[buf, j]],
                                  ssem[buf]).wait()

    issue_load(start, 0)

    def t_body(t, carry):
        for half in range(2):
            k = 2 * t + half
            buf = half

            @pl.when(k < nblk)
            def _():
                wait_load(start + k, buf)
                issue_scat(buf)

            @pl.when(k + 1 < nblk)
            def _():
                @pl.when(k >= 1)
                def __():
                    wait_scat(1 - buf)

                issue_load(start + k + 1, 1 - buf)

        return carry

    lax.fori_loop(0, T_OUTER, t_body, 0)
    wait_scat(0)
    wait_scat(1)
    plsc.subcore_barrier()

    for j in range(ROWS_PER_SUB // CHUNK):
        pltpu.sync_copy(acc.at[pl.ds(base + j * CHUNK, CHUNK)],
                        rows_v.at[j % 2])
        pltpu.sync_copy(rows_v.at[j % 2],
                        psum_hbm.at[cid, pl.ds(base + j * CHUNK, CHUNK)])
    for j in range(ROWS_PER_SUB // CHUNK):
        pltpu.sync_copy(cacc.at[pl.ds(base + j * CHUNK, CHUNK)], zc16_v)
        pltpu.sync_copy(zc16_v, cnt_hbm.at[cid, pl.ds(base + j * CHUNK, CHUNK)])


def _segment_sums(h, b2d, zrow, zcnt, ones):
    mesh = plsc.VectorSubcoreMesh(core_axis_name="c", subcore_axis_name="s")
    return pl.kernel(
        _sc_body,
        out_type=[
            jax.ShapeDtypeStruct((NC, SP, D), jnp.float32),
            jax.ShapeDtypeStruct((NC, SP, 16), jnp.float32),
        ],
        mesh=mesh,
        compiler_params=pltpu.CompilerParams(use_tc_tiling_on_sc=False),
        scratch_types=[
            pltpu.VMEM_SHARED((SP, D), jnp.float32),
            pltpu.VMEM_SHARED((SP, 16), jnp.float32),
            pltpu.VMEM((2, SUB, CHUNK), jnp.int32),
            pltpu.VMEM((2, BLK, D), jnp.float32),
            pltpu.VMEM((CHUNK, 16), jnp.float32),
            pltpu.VMEM((CHUNK, 16), jnp.float32),
            pltpu.SemaphoreType.DMA,
            pltpu.SemaphoreType.DMA,
            pltpu.SemaphoreType.DMA,
            pltpu.SemaphoreType.DMA,
        ],
    )(h, b2d, zrow, zcnt, ones)


def _combine_body(p_ref, c_ref, o_ref):
    cnt = jnp.maximum(c_ref[0, :S, 0:1] + c_ref[1, :S, 0:1], 1.0)
    o_ref[...] = (p_ref[0, :S] + p_ref[1, :S]) / cnt


def _combine(psum, cnt):
    return pl.pallas_call(
        _combine_body,
        out_shape=jax.ShapeDtypeStruct((S, D), jnp.float32),
    )(psum, cnt)


def kernel(x, batch, W, b, gamma, beta):
    wt = W.T
    b2 = b.reshape(1, D)
    g2 = gamma.reshape(1, D)
    bt2 = beta.reshape(1, D)
    h = _linear_ln(x, wt, b2, g2, bt2)
    zrow = jnp.zeros((BLK, D), jnp.float32)
    zcnt = jnp.zeros((CHUNK, 16), jnp.float32)
    ones = jnp.ones((CHUNK, 16), jnp.float32)
    psum, cnt = _segment_sums(h, batch, zrow, zcnt, ones)
    return _combine(psum, cnt)


# speedup vs baseline: 5.8721x; 1.0474x over previous
import jax
import jax.numpy as jnp
from jax import lax
from jax.experimental import pallas as pl
from jax.experimental.pallas import tpu as pltpu
from jax.experimental.pallas import tpu_sc as plsc

N = 320000
D = 128
S = 10000
EPS = 1e-5

ROW_BLOCK = 16000
CHUNK = 128
NUM_CHUNKS = N // CHUNK
NC = 2
NS = 16
NW = NC * NS
SP = 10240
ROWS_PER_SUB = SP // NS


def _linear_ln_body(x_ref, wt_ref, b_ref, g_ref, bt_ref, h_ref):
    h = jnp.dot(x_ref[...], wt_ref[...], preferred_element_type=jnp.float32)
    h = h + b_ref[...]
    mu = jnp.mean(h, axis=-1, keepdims=True)
    var = jnp.mean((h - mu) ** 2, axis=-1, keepdims=True)
    h_ref[...] = (h - mu) * lax.rsqrt(var + EPS) * g_ref[...] + bt_ref[...]


def _linear_ln(x, wt, b2, g2, bt2):
    grid = (N // ROW_BLOCK,)
    return pl.pallas_call(
        _linear_ln_body,
        grid=grid,
        in_specs=[
            pl.BlockSpec((ROW_BLOCK, D), lambda i: (i, 0)),
            pl.BlockSpec((D, D), lambda i: (0, 0)),
            pl.BlockSpec((1, D), lambda i: (0, 0)),
            pl.BlockSpec((1, D), lambda i: (0, 0)),
            pl.BlockSpec((1, D), lambda i: (0, 0)),
        ],
        out_specs=pl.BlockSpec((ROW_BLOCK, D), lambda i: (i, 0)),
        out_shape=jax.ShapeDtypeStruct((N, D), jnp.float32),
    )(x, wt, b2, g2, bt2)


BLK = 128
SUB = BLK // CHUNK
NBLK = N // BLK
T_OUTER = 40


def _sc_body(h_hbm, b2d_hbm, zrow_hbm, zcnt_hbm, ones_hbm, psum_hbm, cnt_hbm,
             acc, cacc, idx_v, rows_v, ones_v, zc16_v,
             lsem0, lsem1, ssem0, ssem1):
    cid = lax.axis_index("c")
    sid = lax.axis_index("s")
    wid = cid * NS + sid
    base = sid * ROWS_PER_SUB
    lsem = (lsem0, lsem1)
    ssem = (ssem0, ssem1)

    start = 78 * wid + jnp.minimum(wid, 4)
    nblk = 78 + jnp.where(wid < 4, 1, 0)

    pltpu.sync_copy(zrow_hbm, rows_v.at[0])
    pltpu.sync_copy(zcnt_hbm, zc16_v)
    pltpu.sync_copy(ones_hbm, ones_v)
    for j in range(ROWS_PER_SUB // CHUNK):
        pltpu.sync_copy(rows_v.at[0], acc.at[pl.ds(base + j * CHUNK, CHUNK)])
    for j in range(ROWS_PER_SUB // CHUNK):
        pltpu.sync_copy(zc16_v, cacc.at[pl.ds(base + j * CHUNK, CHUNK)])
    plsc.subcore_barrier()

    def issue_load(blk, buf):
        pltpu.async_copy(b2d_hbm.at[pl.ds(blk * CHUNK, CHUNK)],
                         idx_v.at[buf, 0], lsem[buf])
        pltpu.async_copy(h_hbm.at[pl.ds(blk * BLK, BLK)], rows_v.at[buf],
                         lsem[buf])

    def wait_load(blk, buf):
        pltpu.make_async_copy(b2d_hbm.at[pl.ds(blk * CHUNK, CHUNK)],
                              idx_v.at[buf, 0], lsem[buf]).wait()
        pltpu.make_async_copy(h_hbm.at[pl.ds(blk * BLK, BLK)], rows_v.at[buf],
                              lsem[buf]).wait()

    def issue_scat(buf):
        for j in range(SUB):
            pltpu.async_copy(rows_v.at[buf, pl.ds(j * CHUNK, CHUNK)],
                             acc.at[idx_v.at[buf, j]], ssem[buf], add=True)
            pltpu.async_copy(ones_v, cacc.at[idx_v.at[buf, j]], ssem[buf],
                             add=True)

    def wait_scat(buf):
        for j in range(SUB):
            pltpu.make_async_copy(rows_v.at[buf, pl.ds(j * CHUNK, CHUNK)],
                                  acc.at[idx_v.at[buf, j]], ssem[buf]).wait()
            pltpu.make_async_copy(ones_v, cacc.at[idx_v.at[buf, j]],
                                  ssem[buf]).wait()

    issue_load(start, 0)

    def t_body(t, carry):
        for half in range(2):
            k = 2 * t + half
            buf = half

            @pl.when(k < nblk)
            def _():
                wait_load(start + k, buf)
                issue_scat(buf)

            @pl.when(k + 1 < nblk)
            def _():
                @pl.when(k >= 1)
                def __():
                    wait_scat(1 - buf)

                issue_load(start + k + 1, 1 - buf)

        return carry

    lax.fori_loop(0, T_OUTER, t_body, 0)
    wait_scat(0)
    wait_scat(1)
    plsc.subcore_barrier()

    for j in range(ROWS_PER_SUB // CHUNK):
        pltpu.sync_copy(acc.at[pl.ds(base + j * CHUNK, CHUNK)],
                        rows_v.at[j % 2])
        pltpu.sync_copy(rows_v.at[j % 2],
                        psum_hbm.at[cid, pl.ds(base + j * CHUNK, CHUNK)])
    for j in range(ROWS_PER_SUB // CHUNK):
        pltpu.sync_copy(cacc.at[pl.ds(base + j * CHUNK, CHUNK)], zc16_v)
        pltpu.sync_copy(zc16_v, cnt_hbm.at[cid, pl.ds(base + j * CHUNK, CHUNK)])


def _segment_sums(h, b2d, zrow, zcnt, ones):
    mesh = plsc.VectorSubcoreMesh(core_axis_name="c", subcore_axis_name="s")
    return pl.kernel(
        _sc_body,
        out_type=[
            jax.ShapeDtypeStruct((NC, SP, D), jnp.float32),
            jax.ShapeDtypeStruct((NC, SP, 16), jnp.float32),
        ],
        mesh=mesh,
        compiler_params=pltpu.CompilerParams(use_tc_tiling_on_sc=False),
        scratch_types=[
            pltpu.VMEM_SHARED((SP, D), jnp.float32),
            pltpu.VMEM_SHARED((SP, 16), jnp.float32),
            pltpu.VMEM((2, SUB, CHUNK), jnp.int32),
            pltpu.VMEM((2, BLK, D), jnp.float32),
            pltpu.VMEM((CHUNK, 16), jnp.float32),
            pltpu.VMEM((CHUNK, 16), jnp.float32),
            pltpu.SemaphoreType.DMA,
            pltpu.SemaphoreType.DMA,
            pltpu.SemaphoreType.DMA,
            pltpu.SemaphoreType.DMA,
        ],
    )(h, b2d, zrow, zcnt, ones)


def _combine_body(p_ref, c_ref, o_ref):
    cnt = jnp.maximum(c_ref[0, :S, 0:1] + c_ref[1, :S, 0:1], 1.0)
    o_ref[...] = (p_ref[0, :S] + p_ref[1, :S]) / cnt


def _combine(psum, cnt):
    return pl.pallas_call(
        _combine_body,
        out_shape=jax.ShapeDtypeStruct((S, D), jnp.float32),
    )(psum, cnt)


def kernel(x, batch, W, b, gamma, beta):
    wt = W.T
    b2 = b.reshape(1, D)
    g2 = gamma.reshape(1, D)
    bt2 = beta.reshape(1, D)
    h = _linear_ln(x, wt, b2, g2, bt2)
    zrow = jnp.zeros((BLK, D), jnp.float32)
    zcnt = jnp.zeros((CHUNK, 16), jnp.float32)
    ones = jnp.ones((CHUNK, 16), jnp.float32)
    psum, cnt = _segment_sums(h, batch, zrow, zcnt, ones)
    return _combine(psum, cnt)
